# baseline (speedup 1.0000x reference)
import jax
import jax.numpy as jnp
from jax import lax
from jax.experimental import pallas as pl
from jax.experimental.pallas import tpu as pltpu
from jax.experimental.pallas import tpu_sc as plsc

N, E, L, D, B = 10000, 320000, 3200000, 128, 9
BPAD = 16
NC, NS = 2, 16
NW = NC * NS
SEG_PER = E // NW
CHUNK = 1024
EDGE_CHUNK = 1000


def _atoms_body(x_ref, w_ref, b_ref, o_ref):
    x = jnp.dot(x_ref[...], w_ref[...], preferred_element_type=jnp.float32)
    o_ref[...] = jax.nn.sigmoid(x + b_ref[...][0:1, :])


def _compute_atoms(node_feat, w16, b16):
    blk = 1000
    return pl.pallas_call(
        _atoms_body,
        grid=(N // blk,),
        in_specs=[
            pl.BlockSpec((blk, D), lambda i: (i, 0)),
            pl.BlockSpec((D, BPAD), lambda i: (0, 0)),
            pl.BlockSpec((8, BPAD), lambda i: (0, 0)),
        ],
        out_specs=pl.BlockSpec((blk, BPAD), lambda i: (i, 0)),
        out_shape=jax.ShapeDtypeStruct((N, BPAD), jnp.float32),
    )(node_feat, w16, b16)


def _edge_gather_body(atoms_hbm, gdst_hbm, out_hbm, idx_v, rows_v, sem):
    c = lax.axis_index("c")
    s = lax.axis_index("s")
    w = s * NC + c
    base = w * (E // NW)
    for i in range(E // NW // EDGE_CHUNK):
        off = base + i * EDGE_CHUNK
        pltpu.sync_copy(gdst_hbm.at[pl.ds(off, EDGE_CHUNK)], idx_v)
        pltpu.async_copy(atoms_hbm.at[idx_v], rows_v, sem).wait()
        pltpu.sync_copy(rows_v, out_hbm.at[pl.ds(off, EDGE_CHUNK)])


def _edge_gather(atoms16, graph_dst):
    return pl.kernel(
        _edge_gather_body,
        out_type=jax.ShapeDtypeStruct((E, BPAD), jnp.float32),
        mesh=plsc.VectorSubcoreMesh(core_axis_name="c", subcore_axis_name="s"),
        scratch_types=[
            pltpu.VMEM((EDGE_CHUNK,), jnp.int32),
            pltpu.VMEM((EDGE_CHUNK, BPAD), jnp.float32),
            pltpu.SemaphoreType.DMA,
        ],
        compiler_params=pltpu.CompilerParams(use_tc_tiling_on_sc=False),
    )(atoms16, graph_dst)


def _segsum_body(tbt_hbm, lg_hbm, seg_hbm, ea_hbm, cuts_hbm, out_hbm,
                 acc_v, tb_v, ea_v, lg_v, seg_v, cuts_v, sem):
    c = lax.axis_index("c")
    s = lax.axis_index("s")
    w = s * NC + c
    pltpu.sync_copy(cuts_hbm, cuts_v)
    lo = jnp.max(plsc.load_gather(cuts_v, [jnp.full((16,), w, jnp.int32)]))
    hi = jnp.max(plsc.load_gather(cuts_v, [jnp.full((16,), w + 1, jnp.int32)]))
    seg_base = w * SEG_PER

    zeros16 = jnp.zeros((16,), jnp.float32)

    for j in range(B):
        @plsc.parallel_loop(0, SEG_PER // 16, unroll=8)
        def _(i):
            acc_v[j, pl.ds(i * 16, 16)] = zeros16

    iota16 = lax.iota(jnp.int32, 16)

    def chunk_body(k, _):
        off = k * CHUNK
        pltpu.sync_copy(lg_hbm.at[pl.ds(off, CHUNK)], lg_v)
        gather = pltpu.async_copy(ea_hbm.at[lg_v], ea_v, sem)
        pltpu.sync_copy(seg_hbm.at[pl.ds(off, CHUNK)], seg_v)
        pltpu.sync_copy(tbt_hbm.at[:, pl.ds(off, CHUNK)], tb_v)
        gather.wait()

        @plsc.parallel_loop(0, CHUNK // 16, unroll=2)
        def _(g):
            rows = g * 16 + iota16
            seg16 = seg_v[pl.ds(g * 16, 16)]
            absi = off + rows
            m = (absi >= lo) & (absi < hi)
            local = jnp.clip(seg16 - seg_base, 0, SEG_PER - 1)
            for j in range(B):
                tbj = tb_v[j, pl.ds(g * 16, 16)]
                eaj = plsc.load_gather(
                    ea_v, [rows, jnp.full((16,), j, jnp.int32)])
                prod = jnp.where(m, tbj * eaj, 0.0)
                plsc.addupdate_scatter(
                    acc_v, [jnp.full((16,), j, jnp.int32), local], prod)

        return 0

    lax.fori_loop(lo // CHUNK, (hi + CHUNK - 1) // CHUNK, chunk_body, 0)
    pltpu.sync_copy(acc_v, out_hbm.at[:, pl.ds(w * SEG_PER, SEG_PER)])


def _segsum(tbt, lg_dst, segment_ids, edge_atoms, cuts):
    return pl.kernel(
        _segsum_body,
        out_type=jax.ShapeDtypeStruct((B, E), jnp.float32),
        mesh=plsc.VectorSubcoreMesh(core_axis_name="c", subcore_axis_name="s"),
        scratch_types=[
            pltpu.VMEM((B, SEG_PER), jnp.float32),
            pltpu.VMEM((B, CHUNK), jnp.float32),
            pltpu.VMEM((CHUNK, BPAD), jnp.float32),
            pltpu.VMEM((CHUNK,), jnp.int32),
            pltpu.VMEM((CHUNK,), jnp.int32),
            pltpu.VMEM((40,), jnp.int32),
            pltpu.SemaphoreType.DMA,
        ],
        compiler_params=pltpu.CompilerParams(use_tc_tiling_on_sc=False,
                                             needs_layout_passes=False),
    )(tbt, lg_dst, segment_ids, edge_atoms, cuts)


def _mlp_body(nbt_ref, ef_ref, w1_ref, b1_ref, wg_ref, bg_ref, o_ref):
    nbt = nbt_ref[...]
    dn = (((0,), (0,)), ((), ()))
    x = lax.dot_general(nbt, w1_ref[...], dn,
                        preferred_element_type=jnp.float32)
    x = x + b1_ref[...][0:1, :]
    g = lax.dot_general(nbt, wg_ref[...], dn,
                        preferred_element_type=jnp.float32)
    g = g + bg_ref[...][0:1, :]
    o_ref[...] = ef_ref[...] + (x * jax.nn.sigmoid(x)) * jax.nn.sigmoid(g)


def _mlp(nbt, edge_feat, w1, b1t, wg, bgt):
    blk = 1280
    return pl.pallas_call(
        _mlp_body,
        grid=(E // blk,),
        in_specs=[
            pl.BlockSpec((B, blk), lambda i: (0, i)),
            pl.BlockSpec((blk, D), lambda i: (i, 0)),
            pl.BlockSpec((B, D), lambda i: (0, 0)),
            pl.BlockSpec((8, D), lambda i: (0, 0)),
            pl.BlockSpec((B, D), lambda i: (0, 0)),
            pl.BlockSpec((8, D), lambda i: (0, 0)),
        ],
        out_specs=pl.BlockSpec((blk, D), lambda i: (i, 0)),
        out_shape=jax.ShapeDtypeStruct((E, D), jnp.float32),
    )(nbt, edge_feat, w1, b1t, wg, bgt)


def kernel(node_feat, edge_feat, graph_dst, lg_src, lg_dst, three_basis,
           three_cutoff, segment_ids, W_atom, b_atom, W1, b1, Wg, bg):
    w16 = jnp.pad(W_atom, ((0, 0), (0, BPAD - B)))
    b16 = jnp.tile(jnp.pad(b_atom, (0, BPAD - B))[None, :], (8, 1))
    atoms16 = _compute_atoms(node_feat, w16, b16)

    edge_atoms = _edge_gather(atoms16, graph_dst.astype(jnp.int32))

    cuts = jnp.searchsorted(
        segment_ids, jnp.arange(0, E + 1, SEG_PER)).astype(jnp.int32)
    cuts = jnp.pad(cuts, (0, 7))
    nbt = _segsum(three_basis.T,
                  lg_dst.astype(jnp.int32),
                  segment_ids.astype(jnp.int32),
                  edge_atoms, cuts)

    b1t = jnp.tile(b1[None, :], (8, 1))
    bgt = jnp.tile(bg[None, :], (8, 1))
    return _mlp(nbt, edge_feat, W1, b1t, Wg, bgt)


# speedup vs baseline: 1.9263x; 1.9263x over previous
import jax
import jax.numpy as jnp
from jax import lax
from jax.experimental import pallas as pl
from jax.experimental.pallas import tpu as pltpu
from jax.experimental.pallas import tpu_sc as plsc

N, E, L, D, B = 10000, 320000, 3200000, 128, 9
BPAD = 16
NC, NS = 2, 16
NW = NC * NS
SEG_PER = E // NW
CHUNK = 1024
EDGE_CHUNK = 1000


def _atoms_body(x_ref, w_ref, b_ref, o_ref):
    x = jnp.dot(x_ref[...], w_ref[...], preferred_element_type=jnp.float32)
    o_ref[...] = jax.nn.sigmoid(x + b_ref[...][0:1, :])


def _compute_atoms(node_feat, w16, b16):
    blk = 1000
    return pl.pallas_call(
        _atoms_body,
        grid=(N // blk,),
        in_specs=[
            pl.BlockSpec((blk, D), lambda i: (i, 0)),
            pl.BlockSpec((D, BPAD), lambda i: (0, 0)),
            pl.BlockSpec((8, BPAD), lambda i: (0, 0)),
        ],
        out_specs=pl.BlockSpec((blk, BPAD), lambda i: (i, 0)),
        out_shape=jax.ShapeDtypeStruct((N, BPAD), jnp.float32),
    )(node_feat, w16, b16)


def _tbt_body(x_ref, o_ref):
    o_ref[...] = x_ref[...].reshape(o_ref.shape)


def _tbt_compact(tbt3):
    blk = 128000
    ni = L // blk
    return pl.pallas_call(
        _tbt_body,
        grid=(B, ni),
        in_specs=[pl.BlockSpec((1, 1, blk), lambda j, i: (j, 0, i))],
        out_specs=pl.BlockSpec((blk,), lambda j, i: (j * ni + i,)),
        out_shape=jax.ShapeDtypeStruct((B * L,), jnp.float32),
    )(tbt3)


def _edge_gather_body(atoms_hbm, gdst_hbm, out_hbm, idx_v, rows_v, sem):
    c = lax.axis_index("c")
    s = lax.axis_index("s")
    w = s * NC + c
    base = w * (E // NW)
    for i in range(E // NW // EDGE_CHUNK):
        off = base + i * EDGE_CHUNK
        pltpu.sync_copy(gdst_hbm.at[pl.ds(off, EDGE_CHUNK)], idx_v)
        pltpu.async_copy(atoms_hbm.at[idx_v], rows_v, sem).wait()
        pltpu.sync_copy(rows_v, out_hbm.at[pl.ds(off, EDGE_CHUNK)])


def _edge_gather(atoms16, graph_dst):
    return pl.kernel(
        _edge_gather_body,
        out_type=jax.ShapeDtypeStruct((E, BPAD), jnp.float32),
        mesh=plsc.VectorSubcoreMesh(core_axis_name="c", subcore_axis_name="s"),
        scratch_types=[
            pltpu.VMEM((EDGE_CHUNK,), jnp.int32),
            pltpu.VMEM((EDGE_CHUNK, BPAD), jnp.float32),
            pltpu.SemaphoreType.DMA,
        ],
        compiler_params=pltpu.CompilerParams(use_tc_tiling_on_sc=False),
    )(atoms16, graph_dst)


def _segsum_body(tbt_hbm, lg_hbm, seg_hbm, ea_hbm, cuts_hbm, out_hbm,
                 acc_v, tb_v, ea_v, lg_v, seg_v, cuts_v, sem):
    c = lax.axis_index("c")
    s = lax.axis_index("s")
    w = s * NC + c
    pltpu.sync_copy(cuts_hbm, cuts_v)
    lo = jnp.max(plsc.load_gather(cuts_v, [jnp.full((16,), w, jnp.int32)]))
    hi = jnp.max(plsc.load_gather(cuts_v, [jnp.full((16,), w + 1, jnp.int32)]))
    seg_base = w * SEG_PER

    zeros16 = jnp.zeros((16,), jnp.float32)

    for j in range(B):
        @plsc.parallel_loop(0, SEG_PER // 16, unroll=8)
        def _(i):
            acc_v[j, pl.ds(i * 16, 16)] = zeros16

    iota16 = lax.iota(jnp.int32, 16)

    def chunk_body(k, _):
        off = k * CHUNK
        pltpu.sync_copy(lg_hbm.at[pl.ds(off, CHUNK)], lg_v)
        gather = pltpu.async_copy(ea_hbm.at[lg_v], ea_v, sem)
        pltpu.sync_copy(seg_hbm.at[pl.ds(off, CHUNK)], seg_v)
        pltpu.sync_copy(tbt_hbm.at[:, pl.ds(off, CHUNK)], tb_v)
        gather.wait()

        @plsc.parallel_loop(0, CHUNK // 16, unroll=2)
        def _(g):
            rows = g * 16 + iota16
            seg16 = seg_v[pl.ds(g * 16, 16)]
            absi = off + rows
            m = (absi >= lo) & (absi < hi)
            local = jnp.clip(seg16 - seg_base, 0, SEG_PER - 1)
            for j in range(B):
                tbj = tb_v[j, pl.ds(g * 16, 16)]
                eaj = plsc.load_gather(
                    ea_v, [rows, jnp.full((16,), j, jnp.int32)])
                prod = jnp.where(m, tbj * eaj, 0.0)
                plsc.addupdate_scatter(
                    acc_v, [jnp.full((16,), j, jnp.int32), local], prod)

        return 0

    lax.fori_loop(lo // CHUNK, (hi + CHUNK - 1) // CHUNK, chunk_body, 0)
    pltpu.sync_copy(acc_v, out_hbm.at[:, pl.ds(w * SEG_PER, SEG_PER)])


def _segsum(tbt, lg_dst, segment_ids, edge_atoms, cuts):
    return pl.kernel(
        _segsum_body,
        out_type=jax.ShapeDtypeStruct((B, E), jnp.float32),
        mesh=plsc.VectorSubcoreMesh(core_axis_name="c", subcore_axis_name="s"),
        scratch_types=[
            pltpu.VMEM((B, SEG_PER), jnp.float32),
            pltpu.VMEM((B, CHUNK), jnp.float32),
            pltpu.VMEM((CHUNK, BPAD), jnp.float32),
            pltpu.VMEM((CHUNK,), jnp.int32),
            pltpu.VMEM((CHUNK,), jnp.int32),
            pltpu.VMEM((40,), jnp.int32),
            pltpu.SemaphoreType.DMA,
        ],
        compiler_params=pltpu.CompilerParams(use_tc_tiling_on_sc=False,
                                             needs_layout_passes=False),
    )(tbt, lg_dst, segment_ids, edge_atoms, cuts)


def _mlp_body(nbt_ref, ef_ref, w1_ref, b1_ref, wg_ref, bg_ref, o_ref):
    nbt = nbt_ref[...]
    dn = (((0,), (0,)), ((), ()))
    x = lax.dot_general(nbt, w1_ref[...], dn,
                        preferred_element_type=jnp.float32)
    x = x + b1_ref[...][0:1, :]
    g = lax.dot_general(nbt, wg_ref[...], dn,
                        preferred_element_type=jnp.float32)
    g = g + bg_ref[...][0:1, :]
    o_ref[...] = ef_ref[...] + (x * jax.nn.sigmoid(x)) * jax.nn.sigmoid(g)


def _mlp(nbt, edge_feat, w1, b1t, wg, bgt):
    blk = 1280
    return pl.pallas_call(
        _mlp_body,
        grid=(E // blk,),
        in_specs=[
            pl.BlockSpec((B, blk), lambda i: (0, i)),
            pl.BlockSpec((blk, D), lambda i: (i, 0)),
            pl.BlockSpec((B, D), lambda i: (0, 0)),
            pl.BlockSpec((8, D), lambda i: (0, 0)),
            pl.BlockSpec((B, D), lambda i: (0, 0)),
            pl.BlockSpec((8, D), lambda i: (0, 0)),
        ],
        out_specs=pl.BlockSpec((blk, D), lambda i: (i, 0)),
        out_shape=jax.ShapeDtypeStruct((E, D), jnp.float32),
    )(nbt, edge_feat, w1, b1t, wg, bgt)


def kernel(node_feat, edge_feat, graph_dst, lg_src, lg_dst, three_basis,
           three_cutoff, segment_ids, W_atom, b_atom, W1, b1, Wg, bg):
    w16 = jnp.pad(W_atom, ((0, 0), (0, BPAD - B)))
    b16 = jnp.tile(jnp.pad(b_atom, (0, BPAD - B))[None, :], (8, 1))
    atoms16 = _compute_atoms(node_feat, w16, b16)

    edge_atoms = _edge_gather(atoms16, graph_dst.astype(jnp.int32))

    cuts = jnp.searchsorted(
        segment_ids, jnp.arange(0, E + 1, SEG_PER)).astype(jnp.int32)
    cuts = jnp.pad(cuts, (0, 7))
    tbt = _tbt_compact(three_basis.T.reshape(B, 1, L)).reshape(B, L)
    nbt = _segsum(tbt,
                  lg_dst.astype(jnp.int32),
                  segment_ids.astype(jnp.int32),
                  edge_atoms, cuts)

    b1t = jnp.tile(b1[None, :], (8, 1))
    bgt = jnp.tile(bg[None, :], (8, 1))
    return _mlp(nbt, edge_feat, W1, b1t, Wg, bgt)


# speedup vs baseline: 2.2055x; 1.1449x over previous
import jax
import jax.numpy as jnp
from jax import lax
from jax.experimental import pallas as pl
from jax.experimental.pallas import tpu as pltpu
from jax.experimental.pallas import tpu_sc as plsc

N, E, L, D, B = 10000, 320000, 3200000, 128, 9
BPAD = 16
NC, NS = 2, 16
NW = NC * NS
SEG_PER = E // NW
CHUNK = 400
NCHUNKS = L // CHUNK
EDGE_CHUNK = 1000


def _atoms_body(x_ref, w_ref, b_ref, o_ref):
    x = jnp.dot(x_ref[...], w_ref[...], preferred_element_type=jnp.float32)
    o_ref[...] = jax.nn.sigmoid(x + b_ref[...][0:1, :])


def _compute_atoms(node_feat, w, bt):
    blk = 1000
    return pl.pallas_call(
        _atoms_body,
        grid=(N // blk,),
        in_specs=[
            pl.BlockSpec((blk, D), lambda i: (i, 0)),
            pl.BlockSpec((D, BPAD), lambda i: (0, 0)),
            pl.BlockSpec((8, BPAD), lambda i: (0, 0)),
        ],
        out_specs=pl.BlockSpec((blk, BPAD), lambda i: (i, 0)),
        out_shape=jax.ShapeDtypeStruct((N, BPAD), jnp.float32),
    )(node_feat, w, bt)


def _tbt_body(x_ref, o_ref):
    o_ref[...] = x_ref[...].reshape(o_ref.shape)


def _tbt_compact(tbt3):
    blk = 128000
    ni = L // blk
    return pl.pallas_call(
        _tbt_body,
        grid=(B, ni),
        in_specs=[pl.BlockSpec((1, 1, blk), lambda j, i: (j, 0, i))],
        out_specs=pl.BlockSpec((blk,), lambda j, i: (j * ni + i,)),
        out_shape=jax.ShapeDtypeStruct((B * L,), jnp.float32),
    )(tbt3)


def _edge_gather_body(atoms_hbm, gdst_hbm, out_hbm, idx_v, rows_v, sem):
    c = lax.axis_index("c")
    s = lax.axis_index("s")
    w = s * NC + c
    base = w * (E // NW)
    for i in range(E // NW // EDGE_CHUNK):
        off = base + i * EDGE_CHUNK
        pltpu.sync_copy(gdst_hbm.at[pl.ds(off, EDGE_CHUNK)], idx_v)
        pltpu.async_copy(atoms_hbm.at[idx_v], rows_v, sem).wait()
        pltpu.sync_copy(rows_v, out_hbm.at[pl.ds(off, EDGE_CHUNK)])


def _edge_gather(atoms, graph_dst):
    return pl.kernel(
        _edge_gather_body,
        out_type=jax.ShapeDtypeStruct((E, BPAD), jnp.float32),
        mesh=plsc.VectorSubcoreMesh(core_axis_name="c", subcore_axis_name="s"),
        scratch_types=[
            pltpu.VMEM((EDGE_CHUNK,), jnp.int32),
            pltpu.VMEM((EDGE_CHUNK, BPAD), jnp.float32),
            pltpu.SemaphoreType.DMA,
        ],
        compiler_params=pltpu.CompilerParams(use_tc_tiling_on_sc=False),
    )(atoms, graph_dst)


def _segsum_body(tbt_hbm, lg_hbm, seg_hbm, ea_hbm, cuts_hbm, out_hbm,
                 acc_v, tb0, tb1, tb2, ea0, ea1, ea2, lg0, lg1, lg2,
                 sg0, sg1, sg2, cuts_v, si0, si1, si2, se0, se1, se2):
    tbs, eas, lgs, sgs = [tb0, tb1, tb2], [ea0, ea1, ea2], \
        [lg0, lg1, lg2], [sg0, sg1, sg2]
    sin, sea = [si0, si1, si2], [se0, se1, se2]

    c = lax.axis_index("c")
    s = lax.axis_index("s")
    w = s * NC + c
    pltpu.sync_copy(cuts_hbm, cuts_v)
    lo = jnp.max(plsc.load_gather(cuts_v, [jnp.full((16,), w, jnp.int32)]))
    hi = jnp.max(plsc.load_gather(cuts_v, [jnp.full((16,), w + 1, jnp.int32)]))
    seg_base = w * SEG_PER

    zeros16 = jnp.zeros((16,), jnp.float32)

    for j in range(B):
        @plsc.parallel_loop(0, SEG_PER // 16, unroll=8)
        def _(i):
            acc_v[j, pl.ds(i * 16, 16)] = zeros16

    iota16 = lax.iota(jnp.int32, 16)

    def clamp(k):
        return jnp.minimum(k, NCHUNKS - 1)

    def issue_in(i, k):
        off = clamp(k) * CHUNK
        pltpu.async_copy(lg_hbm.at[pl.ds(off, CHUNK)], lgs[i], sin[i])
        pltpu.async_copy(seg_hbm.at[pl.ds(off, CHUNK)], sgs[i], sin[i])
        pltpu.async_copy(tbt_hbm.at[:, pl.ds(off, CHUNK)], tbs[i], sin[i])

    def wait_in(i, k):
        off = clamp(k) * CHUNK
        pltpu.make_async_copy(
            lg_hbm.at[pl.ds(off, CHUNK)], lgs[i], sin[i]).wait()
        pltpu.make_async_copy(
            seg_hbm.at[pl.ds(off, CHUNK)], sgs[i], sin[i]).wait()
        pltpu.make_async_copy(
            tbt_hbm.at[:, pl.ds(off, CHUNK)], tbs[i], sin[i]).wait()

    def start_ea(i):
        pltpu.async_copy(ea_hbm.at[lgs[i]], eas[i], sea[i])

    def wait_ea(i):
        pltpu.make_async_copy(ea_hbm.at[lgs[i]], eas[i], sea[i]).wait()

    def compute(i, x):
        off = x * CHUNK

        @plsc.parallel_loop(0, CHUNK // 16, unroll=2)
        def _(g):
            rows = g * 16 + iota16
            seg16 = sgs[i][pl.ds(g * 16, 16)]
            absi = off + rows
            m = (absi >= lo) & (absi < hi)
            local = jnp.clip(seg16 - seg_base, 0, SEG_PER - 1)
            for j in range(B):
                tbj = tbs[i][j, pl.ds(g * 16, 16)]
                eaj = plsc.load_gather(
                    eas[i], [rows, jnp.full((16,), j, jnp.int32)])
                prod = jnp.where(m, tbj * eaj, 0.0)
                plsc.addupdate_scatter(
                    acc_v, [jnp.full((16,), j, jnp.int32), local], prod)

    k0 = lo // CHUNK
    k1 = (hi + CHUNK - 1) // CHUNK
    t1 = (k1 - k0 + 2) // 3

    issue_in(0, k0)
    issue_in(1, k0 + 1)
    wait_in(0, k0)
    start_ea(0)

    def triple_body(t, _):
        x0 = k0 + 3 * t
        for r in range(3):
            x = x0 + r
            wait_in((r + 1) % 3, x + 1)
            start_ea((r + 1) % 3)
            issue_in((r + 2) % 3, x + 2)
            wait_ea(r)
            compute(r, x)
        return 0

    lax.fori_loop(0, t1, triple_body, 0)
    wait_ea(0)
    wait_in(1, k0 + 3 * t1 + 1)
    pltpu.sync_copy(acc_v, out_hbm.at[:, pl.ds(w * SEG_PER, SEG_PER)])


def _segsum(tbt, lg_dst, segment_ids, edge_atoms, cuts):
    return pl.kernel(
        _segsum_body,
        out_type=jax.ShapeDtypeStruct((B, E), jnp.float32),
        mesh=plsc.VectorSubcoreMesh(core_axis_name="c", subcore_axis_name="s"),
        scratch_types=(
            [pltpu.VMEM((B, SEG_PER), jnp.float32)]
            + [pltpu.VMEM((B, CHUNK), jnp.float32) for _ in range(3)]
            + [pltpu.VMEM((CHUNK, BPAD), jnp.float32) for _ in range(3)]
            + [pltpu.VMEM((CHUNK,), jnp.int32) for _ in range(6)]
            + [pltpu.VMEM((40,), jnp.int32)]
            + [pltpu.SemaphoreType.DMA for _ in range(6)]
        ),
        compiler_params=pltpu.CompilerParams(use_tc_tiling_on_sc=False,
                                             needs_layout_passes=False),
    )(tbt, lg_dst, segment_ids, edge_atoms, cuts)


def _mlp_body(nbt_ref, ef_ref, w1_ref, b1_ref, wg_ref, bg_ref, o_ref):
    nbt = nbt_ref[...]
    dn = (((0,), (0,)), ((), ()))
    x = lax.dot_general(nbt, w1_ref[...], dn,
                        preferred_element_type=jnp.float32)
    x = x + b1_ref[...][0:1, :]
    g = lax.dot_general(nbt, wg_ref[...], dn,
                        preferred_element_type=jnp.float32)
    g = g + bg_ref[...][0:1, :]
    o_ref[...] = ef_ref[...] + (x * jax.nn.sigmoid(x)) * jax.nn.sigmoid(g)


def _mlp(nbt, edge_feat, w1, b1t, wg, bgt):
    blk = 1280
    return pl.pallas_call(
        _mlp_body,
        grid=(E // blk,),
        in_specs=[
            pl.BlockSpec((B, blk), lambda i: (0, i)),
            pl.BlockSpec((blk, D), lambda i: (i, 0)),
            pl.BlockSpec((B, D), lambda i: (0, 0)),
            pl.BlockSpec((8, D), lambda i: (0, 0)),
            pl.BlockSpec((B, D), lambda i: (0, 0)),
            pl.BlockSpec((8, D), lambda i: (0, 0)),
        ],
        out_specs=pl.BlockSpec((blk, D), lambda i: (i, 0)),
        out_shape=jax.ShapeDtypeStruct((E, D), jnp.float32),
    )(nbt, edge_feat, w1, b1t, wg, bgt)


def kernel(node_feat, edge_feat, graph_dst, lg_src, lg_dst, three_basis,
           three_cutoff, segment_ids, W_atom, b_atom, W1, b1, Wg, bg):
    w16 = jnp.pad(W_atom, ((0, 0), (0, BPAD - B)))
    bt = jnp.tile(jnp.pad(b_atom, (0, BPAD - B))[None, :], (8, 1))
    atoms = _compute_atoms(node_feat, w16, bt)

    edge_atoms = _edge_gather(atoms, graph_dst.astype(jnp.int32))

    cuts = jnp.searchsorted(
        segment_ids, jnp.arange(0, E + 1, SEG_PER)).astype(jnp.int32)
    cuts = jnp.pad(cuts, (0, 7))
    tbt = _tbt_compact(three_basis.T.reshape(B, 1, L)).reshape(B, L)
    nbt = _segsum(tbt,
                  lg_dst.astype(jnp.int32),
                  segment_ids.astype(jnp.int32),
                  edge_atoms, cuts)

    b1t = jnp.tile(b1[None, :], (8, 1))
    bgt = jnp.tile(bg[None, :], (8, 1))
    return _mlp(nbt, edge_feat, W1, b1t, Wg, bgt)


# speedup vs baseline: 2.2572x; 1.0235x over previous
import jax
import jax.numpy as jnp
from jax import lax
from jax.experimental import pallas as pl
from jax.experimental.pallas import tpu as pltpu
from jax.experimental.pallas import tpu_sc as plsc

N, E, L, D, B = 10000, 320000, 3200000, 128, 9
BPAD = 16
NC, NS = 2, 16
NW = NC * NS
SEG_PER = E // NW
CHUNK = 400
NCHUNKS = L // CHUNK
EDGE_CHUNK = 1000


def _atoms_body(x_ref, w_ref, b_ref, o_ref):
    x = jnp.dot(x_ref[...], w_ref[...], preferred_element_type=jnp.float32)
    o_ref[...] = jax.nn.sigmoid(x + b_ref[...][0:1, :])


def _compute_atoms(node_feat, w, bt):
    blk = 1000
    return pl.pallas_call(
        _atoms_body,
        grid=(N // blk,),
        in_specs=[
            pl.BlockSpec((blk, D), lambda i: (i, 0)),
            pl.BlockSpec((D, BPAD), lambda i: (0, 0)),
            pl.BlockSpec((8, BPAD), lambda i: (0, 0)),
        ],
        out_specs=pl.BlockSpec((blk, BPAD), lambda i: (i, 0)),
        out_shape=jax.ShapeDtypeStruct((N, BPAD), jnp.float32),
    )(node_feat, w, bt)


def _tbt_body(x_ref, o_ref):
    o_ref[...] = x_ref[...].reshape(o_ref.shape)


def _tbt_compact(tbt3):
    blk = 128000
    ni = L // blk
    return pl.pallas_call(
        _tbt_body,
        grid=(B, ni),
        in_specs=[pl.BlockSpec((1, 1, blk), lambda j, i: (j, 0, i))],
        out_specs=pl.BlockSpec((blk,), lambda j, i: (j * ni + i,)),
        out_shape=jax.ShapeDtypeStruct((B * L,), jnp.float32),
    )(tbt3)


def _edge_gather_body(atoms_hbm, gdst_hbm, out_hbm, idx_v, rows_v, sem):
    c = lax.axis_index("c")
    s = lax.axis_index("s")
    w = s * NC + c
    base = w * (E // NW)
    for i in range(E // NW // EDGE_CHUNK):
        off = base + i * EDGE_CHUNK
        pltpu.sync_copy(gdst_hbm.at[pl.ds(off, EDGE_CHUNK)], idx_v)
        pltpu.async_copy(atoms_hbm.at[idx_v], rows_v, sem).wait()
        pltpu.sync_copy(rows_v, out_hbm.at[pl.ds(off, EDGE_CHUNK)])


def _edge_gather(atoms, graph_dst):
    return pl.kernel(
        _edge_gather_body,
        out_type=jax.ShapeDtypeStruct((E, BPAD), jnp.float32),
        mesh=plsc.VectorSubcoreMesh(core_axis_name="c", subcore_axis_name="s"),
        scratch_types=[
            pltpu.VMEM((EDGE_CHUNK,), jnp.int32),
            pltpu.VMEM((EDGE_CHUNK, BPAD), jnp.float32),
            pltpu.SemaphoreType.DMA,
        ],
        compiler_params=pltpu.CompilerParams(use_tc_tiling_on_sc=False),
    )(atoms, graph_dst)


def _segsum_body(tbt_hbm, lg_hbm, seg_hbm, ea_hbm, cuts_hbm, out_hbm,
                 acc_v, tb0, tb1, tb2, ea0, ea1, ea2, lg0, lg1, lg2,
                 sg0, sg1, sg2, cuts_v, si0, si1, si2, se0, se1, se2):
    tbs, eas, lgs, sgs = [tb0, tb1, tb2], [ea0, ea1, ea2], \
        [lg0, lg1, lg2], [sg0, sg1, sg2]
    sin, sea = [si0, si1, si2], [se0, se1, se2]

    c = lax.axis_index("c")
    s = lax.axis_index("s")
    w = s * NC + c
    pltpu.sync_copy(cuts_hbm, cuts_v)
    lo = jnp.max(plsc.load_gather(cuts_v, [jnp.full((16,), w, jnp.int32)]))
    hi = jnp.max(plsc.load_gather(cuts_v, [jnp.full((16,), w + 1, jnp.int32)]))
    seg_base = w * SEG_PER

    zeros16 = jnp.zeros((16,), jnp.float32)

    for j in range(B):
        @plsc.parallel_loop(0, SEG_PER // 16, unroll=8)
        def _(i):
            acc_v[j, pl.ds(i * 16, 16)] = zeros16

    iota16 = lax.iota(jnp.int32, 16)

    def clamp(k):
        return jnp.minimum(k, NCHUNKS - 1)

    def issue_in(i, k):
        off = clamp(k) * CHUNK
        pltpu.async_copy(lg_hbm.at[pl.ds(off, CHUNK)], lgs[i], sin[i])
        pltpu.async_copy(seg_hbm.at[pl.ds(off, CHUNK)], sgs[i], sin[i])
        pltpu.async_copy(tbt_hbm.at[:, pl.ds(off, CHUNK)], tbs[i], sin[i])

    def wait_in(i, k):
        off = clamp(k) * CHUNK
        pltpu.make_async_copy(
            lg_hbm.at[pl.ds(off, CHUNK)], lgs[i], sin[i]).wait()
        pltpu.make_async_copy(
            seg_hbm.at[pl.ds(off, CHUNK)], sgs[i], sin[i]).wait()
        pltpu.make_async_copy(
            tbt_hbm.at[:, pl.ds(off, CHUNK)], tbs[i], sin[i]).wait()

    def start_ea(i):
        pltpu.async_copy(ea_hbm.at[lgs[i]], eas[i], sea[i])

    def wait_ea(i):
        pltpu.make_async_copy(ea_hbm.at[lgs[i]], eas[i], sea[i]).wait()

    def compute(i, x):
        off = x * CHUNK

        @plsc.parallel_loop(0, CHUNK // 16, unroll=5)
        def _(g):
            rows = g * 16 + iota16
            seg16 = sgs[i][pl.ds(g * 16, 16)]
            absi = off + rows
            m = (absi >= lo) & (absi < hi)
            local = jnp.clip(seg16 - seg_base, 0, SEG_PER - 1)
            for j in range(B):
                tbj = tbs[i][j, pl.ds(g * 16, 16)]
                eaj = plsc.load_gather(
                    eas[i], [rows, jnp.full((16,), j, jnp.int32)])
                plsc.addupdate_scatter(
                    acc_v, [jnp.full((16,), j, jnp.int32), local],
                    tbj * eaj, mask=m)

    k0 = lo // CHUNK
    k1 = (hi + CHUNK - 1) // CHUNK
    t1 = (k1 - k0 + 2) // 3

    issue_in(0, k0)
    issue_in(1, k0 + 1)
    wait_in(0, k0)
    start_ea(0)

    def triple_body(t, _):
        x0 = k0 + 3 * t
        for r in range(3):
            x = x0 + r
            wait_in((r + 1) % 3, x + 1)
            start_ea((r + 1) % 3)
            issue_in((r + 2) % 3, x + 2)
            wait_ea(r)
            compute(r, x)
        return 0

    lax.fori_loop(0, t1, triple_body, 0)
    wait_ea(0)
    wait_in(1, k0 + 3 * t1 + 1)
    pltpu.sync_copy(acc_v, out_hbm.at[:, pl.ds(w * SEG_PER, SEG_PER)])


def _segsum(tbt, lg_dst, segment_ids, edge_atoms, cuts):
    return pl.kernel(
        _segsum_body,
        out_type=jax.ShapeDtypeStruct((B, E), jnp.float32),
        mesh=plsc.VectorSubcoreMesh(core_axis_name="c", subcore_axis_name="s"),
        scratch_types=(
            [pltpu.VMEM((B, SEG_PER), jnp.float32)]
            + [pltpu.VMEM((B, CHUNK), jnp.float32) for _ in range(3)]
            + [pltpu.VMEM((CHUNK, BPAD), jnp.float32) for _ in range(3)]
            + [pltpu.VMEM((CHUNK,), jnp.int32) for _ in range(6)]
            + [pltpu.VMEM((40,), jnp.int32)]
            + [pltpu.SemaphoreType.DMA for _ in range(6)]
        ),
        compiler_params=pltpu.CompilerParams(use_tc_tiling_on_sc=False,
                                             needs_layout_passes=False),
    )(tbt, lg_dst, segment_ids, edge_atoms, cuts)


def _mlp_body(nbt_ref, ef_ref, w1_ref, b1_ref, wg_ref, bg_ref, o_ref):
    nbt = nbt_ref[...]
    dn = (((0,), (0,)), ((), ()))
    x = lax.dot_general(nbt, w1_ref[...], dn,
                        preferred_element_type=jnp.float32)
    x = x + b1_ref[...][0:1, :]
    g = lax.dot_general(nbt, wg_ref[...], dn,
                        preferred_element_type=jnp.float32)
    g = g + bg_ref[...][0:1, :]
    o_ref[...] = ef_ref[...] + (x * jax.nn.sigmoid(x)) * jax.nn.sigmoid(g)


def _mlp(nbt, edge_feat, w1, b1t, wg, bgt):
    blk = 2560
    return pl.pallas_call(
        _mlp_body,
        grid=(E // blk,),
        in_specs=[
            pl.BlockSpec((B, blk), lambda i: (0, i)),
            pl.BlockSpec((blk, D), lambda i: (i, 0)),
            pl.BlockSpec((B, D), lambda i: (0, 0)),
            pl.BlockSpec((8, D), lambda i: (0, 0)),
            pl.BlockSpec((B, D), lambda i: (0, 0)),
            pl.BlockSpec((8, D), lambda i: (0, 0)),
        ],
        out_specs=pl.BlockSpec((blk, D), lambda i: (i, 0)),
        out_shape=jax.ShapeDtypeStruct((E, D), jnp.float32),
    )(nbt, edge_feat, w1, b1t, wg, bgt)


def kernel(node_feat, edge_feat, graph_dst, lg_src, lg_dst, three_basis,
           three_cutoff, segment_ids, W_atom, b_atom, W1, b1, Wg, bg):
    w16 = jnp.pad(W_atom, ((0, 0), (0, BPAD - B)))
    bt = jnp.tile(jnp.pad(b_atom, (0, BPAD - B))[None, :], (8, 1))
    atoms = _compute_atoms(node_feat, w16, bt)

    edge_atoms = _edge_gather(atoms, graph_dst.astype(jnp.int32))

    cuts = jnp.searchsorted(
        segment_ids, jnp.arange(0, E + 1, SEG_PER)).astype(jnp.int32)
    cuts = jnp.pad(cuts, (0, 7))
    tbt = _tbt_compact(three_basis.T.reshape(B, 1, L)).reshape(B, L)
    nbt = _segsum(tbt,
                  lg_dst.astype(jnp.int32),
                  segment_ids.astype(jnp.int32),
                  edge_atoms, cuts)

    b1t = jnp.tile(b1[None, :], (8, 1))
    bgt = jnp.tile(bg[None, :], (8, 1))
    return _mlp(nbt, edge_feat, W1, b1t, Wg, bgt)
